# baseline (speedup 1.0000x reference)
import functools

import jax
import jax.numpy as jnp
from jax import lax
from jax.experimental import pallas as pl
from jax.experimental.pallas import tpu as pltpu
from jax.experimental.pallas import tpu_sc as plsc

B, T = 8, 2048
K, CDIM, OUT_DIM = 1024, 1024, 768
N_TOK = B * T
HALF = OUT_DIM // 2

def _proj_body(cent_ref, w_ref, b_ref, p_ref):
    p = lax.dot_general(
        cent_ref[...], w_ref[...],
        dimension_numbers=(((1,), (1,)), ((), ())),
        preferred_element_type=jnp.float32,
    ) + b_ref[...]
    lo = lax.bitcast_convert_type(p[:, :HALF].astype(jnp.bfloat16), jnp.int16)
    hi = lax.bitcast_convert_type(p[:, HALF:].astype(jnp.bfloat16), jnp.int16)
    p_ref[...] = (lo.astype(jnp.int32) & 0xFFFF) | (hi.astype(jnp.int32) << 16)


def _project(centroids, W, b):
    return pl.pallas_call(
        _proj_body,
        out_shape=jax.ShapeDtypeStruct((K, HALF), jnp.int32),
    )(centroids, W, b.reshape(1, OUT_DIM))


_NC, _NS = 2, 16
_NW = _NC * _NS
_ROWS_PER_W = N_TOK // _NW
_CHUNK = 64
_NCHUNK = _ROWS_PER_W // _CHUNK


def _gather_kernel(table_hbm, idx_hbm, out_hbm, idx_v, buf0, buf1, sem0, sem1):
    wid = lax.axis_index("s") * _NC + lax.axis_index("c")
    base = wid * _ROWS_PER_W
    pltpu.sync_copy(idx_hbm.at[pl.ds(base, _ROWS_PER_W)], idx_v)
    bufs = (buf0, buf1)
    sems = (sem0, sem1)
    copies = [None, None]
    copies[0] = pltpu.async_copy(
        table_hbm.at[idx_v.at[pl.ds(0, _CHUNK)]], bufs[0], sems[0])
    for k in range(_NCHUNK):
        cur = k % 2
        nxt = (k + 1) % 2
        if k + 1 < _NCHUNK:
            copies[nxt] = pltpu.async_copy(
                table_hbm.at[idx_v.at[pl.ds((k + 1) * _CHUNK, _CHUNK)]],
                bufs[nxt], sems[nxt])
        copies[cur].wait()
        pltpu.sync_copy(bufs[cur], out_hbm.at[pl.ds(base + k * _CHUNK, _CHUNK)])


def _gather(table, idx):
    mesh = plsc.VectorSubcoreMesh(core_axis_name="c", subcore_axis_name="s")
    return pl.kernel(
        _gather_kernel,
        mesh=mesh,
        out_type=jax.ShapeDtypeStruct((N_TOK, HALF), jnp.int32),
        scratch_types=[
            pltpu.VMEM((_ROWS_PER_W,), jnp.int32),
            pltpu.VMEM((_CHUNK, HALF), jnp.int32),
            pltpu.VMEM((_CHUNK, HALF), jnp.int32),
            pltpu.SemaphoreType.DMA,
            pltpu.SemaphoreType.DMA,
        ],
    )(table, idx)


_TBLK = 512


def _unpack(words):
    lo16 = ((words << 16) >> 16).astype(jnp.int16)
    return lax.bitcast_convert_type(lo16, jnp.bfloat16).astype(jnp.float32)


def _tpose_body(g_ref, o_ref):
    gt = g_ref[...].T
    o_ref[0, :HALF] = _unpack(gt)
    o_ref[0, HALF:] = _unpack(gt >> 16)


def _transpose(g):
    nblk = T // _TBLK
    return pl.pallas_call(
        _tpose_body,
        grid=(B * nblk,),
        in_specs=[pl.BlockSpec((_TBLK, HALF), lambda i: (i, 0))],
        out_specs=pl.BlockSpec(
            (1, OUT_DIM, _TBLK), lambda i: (i // nblk, 0, i % nblk)),
        out_shape=jax.ShapeDtypeStruct((B, OUT_DIM, T), jnp.float32),
        compiler_params=pltpu.CompilerParams(
            dimension_semantics=("parallel",)),
    )(g)


def kernel(c, centroids, W, b):
    proj_table = _project(centroids, W, b)
    idx = c.reshape(-1).astype(jnp.int32)
    gathered = _gather(proj_table, idx)
    return _transpose(gathered)


# speedup vs baseline: 1.1749x; 1.1749x over previous
import functools

import jax
import jax.numpy as jnp
from jax import lax
from jax.experimental import pallas as pl
from jax.experimental.pallas import tpu as pltpu
from jax.experimental.pallas import tpu_sc as plsc

B, T = 8, 2048
K, CDIM, OUT_DIM = 1024, 1024, 768
N_TOK = B * T
HALF = OUT_DIM // 2

def _proj_body(cent_ref, w_ref, b_ref, p_ref):
    p = lax.dot_general(
        cent_ref[...], w_ref[...],
        dimension_numbers=(((1,), (1,)), ((), ())),
        preferred_element_type=jnp.float32,
    ) + b_ref[...]
    lo = lax.bitcast_convert_type(p[:, :HALF].astype(jnp.bfloat16), jnp.int16)
    hi = lax.bitcast_convert_type(p[:, HALF:].astype(jnp.bfloat16), jnp.int16)
    p_ref[...] = (lo.astype(jnp.int32) & 0xFFFF) | (hi.astype(jnp.int32) << 16)


def _project(centroids, W, b):
    return pl.pallas_call(
        _proj_body,
        out_shape=jax.ShapeDtypeStruct((K, HALF), jnp.int32),
    )(centroids, W, b.reshape(1, OUT_DIM))


_NC, _NS = 2, 16
_NW = _NC * _NS
_ROWS_PER_W = N_TOK // _NW
_CHUNK = 64
_NCHUNK = _ROWS_PER_W // _CHUNK


def _gather_kernel(table_hbm, idx_hbm, out_hbm, idx_v, buf0, buf1, sem0, sem1):
    wid = lax.axis_index("s") * _NC + lax.axis_index("c")
    base = wid * _ROWS_PER_W
    pltpu.sync_copy(idx_hbm.at[pl.ds(base, _ROWS_PER_W)], idx_v)
    bufs = (buf0, buf1)
    sems = (sem0, sem1)
    copies = [None, None]
    copies[0] = pltpu.async_copy(
        table_hbm.at[idx_v.at[pl.ds(0, _CHUNK)]], bufs[0], sems[0])
    for k in range(_NCHUNK):
        cur = k % 2
        nxt = (k + 1) % 2
        if k + 1 < _NCHUNK:
            copies[nxt] = pltpu.async_copy(
                table_hbm.at[idx_v.at[pl.ds((k + 1) * _CHUNK, _CHUNK)]],
                bufs[nxt], sems[nxt])
        copies[cur].wait()
        pltpu.sync_copy(bufs[cur], out_hbm.at[pl.ds(base + k * _CHUNK, _CHUNK)])


def _gather(table, idx):
    mesh = plsc.VectorSubcoreMesh(core_axis_name="c", subcore_axis_name="s")
    return pl.kernel(
        _gather_kernel,
        mesh=mesh,
        out_type=jax.ShapeDtypeStruct((N_TOK, HALF), jnp.int32),
        scratch_types=[
            pltpu.VMEM((_ROWS_PER_W,), jnp.int32),
            pltpu.VMEM((_CHUNK, HALF), jnp.int32),
            pltpu.VMEM((_CHUNK, HALF), jnp.int32),
            pltpu.SemaphoreType.DMA,
            pltpu.SemaphoreType.DMA,
        ],
    )(table, idx)


_TBLK = 2048


def _unpack(words):
    lo16 = ((words << 16) >> 16).astype(jnp.int16)
    return lax.bitcast_convert_type(lo16, jnp.bfloat16).astype(jnp.float32)


def _tpose_body(g_ref, o_ref):
    gt = g_ref[...].T
    o_ref[0, :HALF] = _unpack(gt)
    o_ref[0, HALF:] = _unpack(gt >> 16)


def _transpose(g):
    nblk = T // _TBLK
    return pl.pallas_call(
        _tpose_body,
        grid=(B * nblk,),
        in_specs=[pl.BlockSpec((_TBLK, HALF), lambda i: (i, 0))],
        out_specs=pl.BlockSpec(
            (1, OUT_DIM, _TBLK), lambda i: (i // nblk, 0, i % nblk)),
        out_shape=jax.ShapeDtypeStruct((B, OUT_DIM, T), jnp.float32),
        compiler_params=pltpu.CompilerParams(
            dimension_semantics=("parallel",)),
    )(g)


def kernel(c, centroids, W, b):
    proj_table = _project(centroids, W, b)
    idx = c.reshape(-1).astype(jnp.int32)
    gathered = _gather(proj_table, idx)
    return _transpose(gathered)
